# Initial kernel scaffold; baseline (speedup 1.0000x reference)
import jax
import jax.numpy as jnp
from jax.experimental import pallas as pl


def kernel(agent_pos, landmark_pos, agent_vel, other_pos, relative_landmark_pos, W_l, b_l, W_r, b_r, W_e, att, bias, W1, b1, W2, b2):
    raise NotImplementedError("write your pallas kernel here")


# speedup vs baseline: 211.2904x; 211.2904x over previous
import functools
import jax
import jax.numpy as jnp
from jax.experimental import pallas as pl
from jax.experimental.pallas import tpu as pltpu

B = 16384
TILE = 512
SRC = (0, 0, 1, 1, 2, 2)
DST = (1, 2, 0, 2, 0, 1)
IN_EDGES = ((2, 1, 4, 2), (0, 0, 5, 2), (1, 0, 3, 1))


def _dot(a, b):
    return jax.lax.dot(a, b, preferred_element_type=jnp.float32)


def _tc_body(lmT_ref, featT_ref, WlrT_ref, WeT_ref, att_ref, blr_ref,
             bias_ref, W1T_ref, b1_ref, W2T_ref, b2_ref, outT_ref):
    featT = featT_ref[:, :]
    WlrT = WlrT_ref[:, :]
    blr = blr_ref[:, :]
    bias = bias_ref[:, :]
    att = att_ref[:, :]
    WeT = WeT_ref[:, :]

    xl, xr, pos = [], [], []
    for j in range(3):
        fj = featT[14 * j:14 * j + 14, :]
        xlr = _dot(WlrT, fj) + blr
        xl.append(xlr[0:16, :])
        xr.append(xlr[16:32, :])
        pos.append(fj[0:2, :])

    alphas = []
    for e in range(6):
        s, d = SRC[e], DST[e]
        cx = pos[d][0:1, :] - pos[s][0:1, :]
        cy = pos[d][1:2, :] - pos[s][1:2, :]
        dist = jnp.sqrt(cx * cx + cy * cy)
        eT = WeT[:, 0:1] * cx + WeT[:, 1:2] * cy + WeT[:, 2:3] * dist
        m = xl[s] + xr[d] + eT
        m = jnp.where(m > 0, m, 0.2 * m)
        alphas.append(jnp.sum(m * att, axis=0, keepdims=True))

    pool = jnp.zeros((16, TILE), jnp.float32)
    for d in range(3):
        e1, s1, e2, s2 = IN_EDGES[d]
        a1, a2 = alphas[e1], alphas[e2]
        amax = jnp.maximum(a1, a2)
        x1 = jnp.exp(a1 - amax)
        x2 = jnp.exp(a2 - amax)
        den = x1 + x2 + 1e-16
        o = (x1 / den) * xl[s1] + (x2 / den) * xl[s2] + bias
        pool = pool + jnp.maximum(o, 0.0)

    WlT = WlrT[0:16, :]
    VT = WlT[:, 6:10] + WlT[:, 10:14]
    AT = jnp.concatenate([
        WlT[:, 0:1] - VT[:, 0:1] - VT[:, 2:3],
        WlT[:, 1:2] - VT[:, 1:2] - VT[:, 3:4],
        VT,
    ], axis=1)
    xlobj = _dot(AT, lmT_ref[:, :]) + blr[0:16, :] + bias
    objpool = 3.0 * jnp.maximum(xlobj, 0.0)

    h = jnp.concatenate([pool, objpool], axis=0)
    hid = jnp.maximum(_dot(W1T_ref[:, :], h) + b1_ref[:, :], 0.0)
    outT_ref[:, :] = _dot(W2T_ref[:, :], hid) + b2_ref[:, :]


@jax.jit
def _run(lmT, featT, WlrT, WeT, att2, blr, bias2, W1T, b1, W2T, b2):
    grid = (B // TILE,)
    full = lambda shape: pl.BlockSpec(shape, lambda i: (0, 0))
    outT = pl.pallas_call(
        _tc_body,
        grid=grid,
        in_specs=[
            pl.BlockSpec((6, TILE), lambda i: (0, i)),
            pl.BlockSpec((42, TILE), lambda i: (0, i)),
            full((32, 14)),
            full((16, 3)),
            full((16, 1)),
            full((32, 1)),
            full((16, 1)),
            full((128, 32)),
            full((128, 1)),
            full((32, 128)),
            full((32, 1)),
        ],
        out_specs=pl.BlockSpec((32, TILE), lambda i: (0, i)),
        out_shape=jax.ShapeDtypeStruct((32, B), jnp.float32),
    )(lmT, featT, WlrT, WeT, att2, blr, bias2, W1T, b1, W2T, b2)
    return outT.T


def kernel(agent_pos, landmark_pos, agent_vel, other_pos, relative_landmark_pos,
           W_l, b_l, W_r, b_r, W_e, att, bias, W1, b1, W2, b2):
    b = agent_pos.shape[0]
    lmT = landmark_pos.reshape(b, 6).T
    feat = jnp.concatenate(
        [agent_pos, agent_vel, relative_landmark_pos, other_pos], axis=2)
    featT = feat.reshape(b, 42).T
    WlrT = jnp.concatenate([W_l.T, W_r.T], axis=0)
    blr = jnp.concatenate([b_l, b_r])[:, None]
    return _run(lmT, featT, WlrT, W_e.T, att[:, None], blr, bias[:, None],
                W1.T, b1[:, None], W2.T, b2[:, None])
